# Initial kernel scaffold; baseline (speedup 1.0000x reference)
import jax
import jax.numpy as jnp
from jax.experimental import pallas as pl


def kernel(bbox, x_tab, y_tab, h_tab, w_tab):
    raise NotImplementedError("write your pallas kernel here")


# speedup vs baseline: 1.8371x; 1.8371x over previous
import functools

import jax
import jax.numpy as jnp
from jax import lax
from jax.experimental import pallas as pl
from jax.experimental.pallas import tpu as pltpu
from jax.experimental.pallas import tpu_sc as plsc

LANES = 16
ROWS_PER_CHUNK = 128


def _build_sc_call(T, V, D):
    info = plsc.get_sparse_core_info()
    NC, NS = info.num_cores, info.num_subcores
    NW = NC * NS
    assert T % (NW * LANES) == 0
    TPW = T // NW
    RPW = TPW * 6
    n_chunks = RPW // ROWS_PER_CHUNK
    assert RPW % ROWS_PER_CHUNK == 0
    n_groups = TPW // LANES

    mesh = plsc.VectorSubcoreMesh(core_axis_name="c", subcore_axis_name="s")

    @functools.partial(
        pl.kernel,
        mesh=mesh,
        out_type=jax.ShapeDtypeStruct((T * 6, D), jnp.float32),
        compiler_params=pltpu.CompilerParams(needs_layout_passes=False),
        scratch_types=[
            pltpu.VMEM((TPW * 4,), jnp.int32),
            pltpu.VMEM((RPW,), jnp.int32),
            pltpu.VMEM((ROWS_PER_CHUNK, D), jnp.float32),
            pltpu.SemaphoreType.DMA,
        ],
    )
    def sc_kernel(bbox_hbm, table_hbm, out_hbm, bbox_v, idx_v, rows_v, sem):
        wid = lax.axis_index("s") * NC + lax.axis_index("c")
        tok0 = wid * TPW
        row0 = wid * RPW

        pltpu.sync_copy(bbox_hbm.at[pl.ds(tok0 * 4, TPW * 4)], bbox_v)

        lane = lax.iota(jnp.int32, 16)

        def compute_group(g, carry):
            base4 = (g * LANES + lane) * 4
            x0 = plsc.load_gather(bbox_v, [base4])
            y0 = plsc.load_gather(bbox_v, [base4 + 1])
            x1 = plsc.load_gather(bbox_v, [base4 + 2])
            y1 = plsc.load_gather(bbox_v, [base4 + 3])
            base6 = (g * LANES + lane) * 6
            plsc.store_scatter(idx_v, [base6], x0)
            plsc.store_scatter(idx_v, [base6 + 1], y0 + V)
            plsc.store_scatter(idx_v, [base6 + 2], x1)
            plsc.store_scatter(idx_v, [base6 + 3], y1 + V)
            plsc.store_scatter(idx_v, [base6 + 4], (y1 - y0) + 2 * V)
            plsc.store_scatter(idx_v, [base6 + 5], (x1 - x0) + 3 * V)
            return carry

        lax.fori_loop(0, n_groups, compute_group, 0)

        def do_chunk(ci, carry):
            idx_slice = idx_v.at[pl.ds(ci * ROWS_PER_CHUNK, ROWS_PER_CHUNK)]
            pltpu.async_copy(table_hbm.at[idx_slice], rows_v, sem).wait()
            pltpu.sync_copy(
                rows_v,
                out_hbm.at[pl.ds(row0 + ci * ROWS_PER_CHUNK, ROWS_PER_CHUNK)],
            )
            return carry

        lax.fori_loop(0, n_chunks, do_chunk, 0)

    return sc_kernel


def kernel(bbox, x_tab, y_tab, h_tab, w_tab):
    B, N, _ = bbox.shape
    V, D = x_tab.shape
    T = B * N
    table = jnp.concatenate([x_tab, y_tab, h_tab, w_tab], axis=0)
    bbox_flat = bbox.reshape(T * 4)
    out = _build_sc_call(T, V, D)(bbox_flat, table)
    return out.reshape(B, N, 6 * D)
